# baseline (speedup 1.0000x reference)
import functools

import jax
import jax.numpy as jnp
import numpy as np
from jax import lax
from jax.experimental import pallas as pl
from jax.experimental.pallas import tpu as pltpu
from jax.experimental.pallas import tpu_sc as plsc

B, S, D, E, K = 4, 4096, 4096, 64, 8
SCALE = 1.0 / np.sqrt(D)
N_TOKENS = B * S

ROW_BLOCK = 1024
L = 16
CHUNK = 128


def _bf16_dot(a, b):
    return jax.lax.dot_general(
        a, b,
        dimension_numbers=(((1,), (0,)), ((), ())),
        preferred_element_type=jnp.float32,
    )


def _logits_block_kernel(x_ref, wh_ref, out_ref):
    xh = x_ref[...].astype(jnp.bfloat16)
    out_ref[...] = _bf16_dot(xh, wh_ref[...]) * SCALE


def _tc_logits(x_flat, wh):
    n_rows = x_flat.shape[0]
    grid = (n_rows // ROW_BLOCK,)
    return pl.pallas_call(
        _logits_block_kernel,
        grid=grid,
        in_specs=[
            pl.BlockSpec((ROW_BLOCK, D), lambda i: (i, 0)),
            pl.BlockSpec((D, E), lambda i: (0, 0)),
        ],
        out_specs=pl.BlockSpec((ROW_BLOCK, E), lambda i: (i, 0)),
        out_shape=jax.ShapeDtypeStruct((n_rows, E), jnp.float32),
    )(x_flat, wh)


def _make_sc_router(n_tokens):
    nc, ns = 2, 16
    nw = nc * ns
    tok_per_w = n_tokens // nw
    mesh = plsc.VectorSubcoreMesh(
        core_axis_name="c", subcore_axis_name="s",
        num_cores=nc, num_subcores=ns)

    @functools.partial(
        pl.kernel, mesh=mesh,
        compiler_params=pltpu.CompilerParams(needs_layout_passes=False),
        out_type=jax.ShapeDtypeStruct((n_tokens, L), jnp.float32),
        scratch_types=[
            pltpu.VMEM((CHUNK, E), jnp.float32),
            pltpu.VMEM((CHUNK, L), jnp.float32),
            pltpu.VMEM((E,), jnp.float32),
        ],
    )
    def sc_router(logits_hbm, bias_hbm, out_hbm, logits_v, out_v, bias_v):
        wid = lax.axis_index("s") * nc + lax.axis_index("c")
        base = wid * tok_per_w
        pltpu.sync_copy(bias_hbm, bias_v)

        lane = lax.iota(jnp.int32, L)
        low8 = lane < 8
        dup8 = jnp.where(low8, lane, lane - 8)
        neg_inf = jnp.float32(-jnp.inf)

        bias_parts = []
        for j in range(E // L):
            bias_parts.append(bias_v[pl.ds(j * L, L)])

        def gather16(v, idx):
            return lax.gather(
                v, idx.reshape(L, 1),
                lax.GatherDimensionNumbers(
                    offset_dims=(), collapsed_slice_dims=(0,),
                    start_index_map=(0,)),
                (1,), mode=lax.GatherScatterMode.PROMISE_IN_BOUNDS)

        def merge8(ak, av, bk, bv):
            ck = jnp.where(low8, ak, gather16(bk, dup8))
            cv = jnp.where(low8, av, gather16(bv, dup8))
            return plsc.sort_key_val(ck, cv, descending=True)

        def body(t, _):
            sk, sv = [], []
            for j in range(E // L):
                v = logits_v[t, pl.ds(j * L, L)] + bias_parts[j]
                idx = lane + (j * L)
                k_, v_ = plsc.sort_key_val(v, idx, descending=True)
                sk.append(k_)
                sv.append(v_)
            k01, v01 = merge8(sk[0], sv[0], sk[1], sv[1])
            k23, v23 = merge8(sk[2], sv[2], sk[3], sv[3])
            kf, vf = merge8(k01, v01, k23, v23)
            em = jnp.bitwise_and(vf, L - 1)
            g0 = gather16(bias_parts[0], em)
            g1 = gather16(bias_parts[1], em)
            g2 = gather16(bias_parts[2], em)
            g3 = gather16(bias_parts[3], em)
            bsel = jnp.where(vf < 16, g0,
                             jnp.where(vf < 32, g1,
                                       jnp.where(vf < 48, g2, g3)))
            orig = kf - bsel
            masked = jnp.where(low8, orig, neg_inf)
            mx = lax.reduce_max(masked, (0,))
            e = jnp.where(low8, jnp.exp(orig - mx), 0.0)
            ssum = lax.reduce_sum(e, (0,))
            w = e / ssum
            packed = jnp.where(
                low8, w,
                gather16(plsc.bitcast(vf, jnp.float32), dup8))
            out_v[t, :] = packed
            return ()

        for c in range(tok_per_w // CHUNK):
            off = base + c * CHUNK
            pltpu.sync_copy(logits_hbm.at[pl.ds(off, CHUNK), :], logits_v)
            lax.fori_loop(0, CHUNK, body, ())
            pltpu.sync_copy(out_v, out_hbm.at[pl.ds(off, CHUNK), :])

    return sc_router


_SC_ROUTERS = {}


def _sc_router(n_tokens):
    if n_tokens not in _SC_ROUTERS:
        _SC_ROUTERS[n_tokens] = _make_sc_router(n_tokens)
    return _SC_ROUTERS[n_tokens]


N_OVERLAP_CHUNKS = 4


@jax.jit
def kernel(x, W, routing_bias):
    x_flat = x.reshape(N_TOKENS, D)
    wt_hi = W.T.astype(jnp.bfloat16)

    tok_chunk = N_TOKENS // N_OVERLAP_CHUNKS
    packs = []
    for c in range(N_OVERLAP_CHUNKS):
        xs = jax.lax.slice_in_dim(x_flat, c * tok_chunk, (c + 1) * tok_chunk)
        logits = _tc_logits(xs, wt_hi)
        packs.append(_sc_router(tok_chunk)(logits, routing_bias))
    packed = jnp.concatenate(packs, axis=0)
    weights = packed[:, :K]
    indices = jax.lax.bitcast_convert_type(packed[:, K:], jnp.int32)
    return (weights.reshape(B, S, K), indices.reshape(B, S, K))


# speedup vs baseline: 2.8659x; 2.8659x over previous
import functools

import jax
import jax.numpy as jnp
import numpy as np
from jax.experimental import pallas as pl

B, S, D, E, K = 4, 4096, 4096, 64, 8
SCALE = 1.0 / np.sqrt(D)

ROW_BLOCK = 1024


def _bf16_dot(a, b):
    return jax.lax.dot_general(
        a, b,
        dimension_numbers=(((1,), (0,)), ((), ())),
        preferred_element_type=jnp.float32,
    )


def _router_block_kernel(x_ref, wh_ref, bias_ref, w_out_ref, i_out_ref):
    xf = x_ref[...]
    xh = xf.astype(jnp.bfloat16)
    dots = _bf16_dot(xh, wh_ref[...])
    orig = jnp.transpose(dots * SCALE)
    biased = orig + bias_ref[...]

    eiota = jax.lax.broadcasted_iota(jnp.int32, biased.shape, 0)
    neg_inf = jnp.float32(-jnp.inf)

    l = biased
    vals = []
    idxs = []
    for _ in range(K):
        m = jnp.max(l, axis=0, keepdims=True)
        idx = jnp.min(jnp.where(l == m, eiota, E), axis=0, keepdims=True)
        hit = eiota == idx
        ov = jnp.sum(jnp.where(hit, orig, 0.0), axis=0, keepdims=True)
        vals.append(ov)
        idxs.append(idx)
        l = jnp.where(hit, neg_inf, l)

    v = jnp.concatenate(vals, axis=0)
    inds = jnp.concatenate(idxs, axis=0)

    mx = jnp.max(v, axis=0, keepdims=True)
    e = jnp.exp(v - mx)
    w = e / jnp.sum(e, axis=0, keepdims=True)

    w_out_ref[...] = jnp.transpose(w)
    i_out_ref[...] = jnp.transpose(inds)


@functools.partial(jax.jit, static_argnames=())
def kernel(x, W, routing_bias):
    n_tokens = B * S
    x_flat = x.reshape(n_tokens, D)
    wt_hi = W.T.astype(jnp.bfloat16)
    bias = routing_bias.reshape(E, 1)

    grid = (n_tokens // ROW_BLOCK,)
    weights_t, indices_t = pl.pallas_call(
        _router_block_kernel,
        grid=grid,
        in_specs=[
            pl.BlockSpec((ROW_BLOCK, D), lambda i: (i, 0)),
            pl.BlockSpec((D, E), lambda i: (0, 0)),
            pl.BlockSpec((E, 1), lambda i: (0, 0)),
        ],
        out_specs=[
            pl.BlockSpec((ROW_BLOCK, K), lambda i: (i, 0)),
            pl.BlockSpec((ROW_BLOCK, K), lambda i: (i, 0)),
        ],
        out_shape=[
            jax.ShapeDtypeStruct((n_tokens, K), jnp.float32),
            jax.ShapeDtypeStruct((n_tokens, K), jnp.int32),
        ],
    )(x_flat, wt_hi, bias)

    return (weights_t.reshape(B, S, K), indices_t.reshape(B, S, K))


# speedup vs baseline: 3.3763x; 1.1781x over previous
import functools

import jax
import jax.numpy as jnp
import numpy as np
from jax.experimental import pallas as pl

B, S, D, E, K = 4, 4096, 4096, 64, 8
SCALE = 1.0 / np.sqrt(D)

ROW_BLOCK = 1024


def _bf16_dot(a, b):
    return jax.lax.dot_general(
        a, b,
        dimension_numbers=(((1,), (0,)), ((), ())),
        preferred_element_type=jnp.float32,
    )


def _router_block_kernel(x_ref, wh_ref, bias_ref, w_out_ref, i_out_ref):
    xf = x_ref[...]
    xh = xf.astype(jnp.bfloat16)
    dots = _bf16_dot(xh, wh_ref[...])
    orig = jnp.transpose(dots * SCALE)
    biased = orig + bias_ref[...]

    eiota = jax.lax.broadcasted_iota(jnp.int32, biased.shape, 0)
    neg_inf = jnp.float32(-jnp.inf)

    l = biased
    vals = []
    idxs = []
    for _ in range(K):
        m = jnp.max(l, axis=0, keepdims=True)
        idx = jnp.min(jnp.where(l == m, eiota, E), axis=0, keepdims=True)
        hit = eiota == idx
        ov = jnp.sum(jnp.where(hit, orig, 0.0), axis=0, keepdims=True)
        vals.append(ov)
        idxs.append(idx)
        l = jnp.where(hit, neg_inf, l)

    v = jnp.concatenate(vals, axis=0)
    inds = jnp.concatenate(idxs, axis=0)

    mx = jnp.max(v, axis=0, keepdims=True)
    e = jnp.exp(v - mx)
    w = e / jnp.sum(e, axis=0, keepdims=True)

    w_out_ref[...] = w
    i_out_ref[...] = inds


@functools.partial(jax.jit, static_argnames=())
def kernel(x, W, routing_bias):
    n_tokens = B * S
    x_flat = x.reshape(n_tokens, D)
    wt_hi = W.T.astype(jnp.bfloat16)
    bias = routing_bias.reshape(E, 1)

    grid = (n_tokens // ROW_BLOCK,)
    weights_t, indices_t = pl.pallas_call(
        _router_block_kernel,
        grid=grid,
        in_specs=[
            pl.BlockSpec((ROW_BLOCK, D), lambda i: (i, 0)),
            pl.BlockSpec((D, E), lambda i: (0, 0)),
            pl.BlockSpec((E, 1), lambda i: (0, 0)),
        ],
        out_specs=[
            pl.BlockSpec((K, ROW_BLOCK), lambda i: (0, i)),
            pl.BlockSpec((K, ROW_BLOCK), lambda i: (0, i)),
        ],
        out_shape=[
            jax.ShapeDtypeStruct((K, n_tokens), jnp.float32),
            jax.ShapeDtypeStruct((K, n_tokens), jnp.int32),
        ],
    )(x_flat, wt_hi, bias)

    return (weights_t.T.reshape(B, S, K), indices_t.T.reshape(B, S, K))
